# baseline (speedup 1.0000x reference)
import jax
import jax.numpy as jnp
from jax.experimental import pallas as pl

_BC = 640_000


def _copy_block(x_ref, o_ref):
    o_ref[...] = x_ref[...]


def kernel(edge_index):
    E = edge_index.shape[1]
    out = pl.pallas_call(
        _copy_block,
        grid=(E // _BC,),
        in_specs=[pl.BlockSpec((2, _BC), lambda i: (0, i))],
        out_specs=pl.BlockSpec((2, _BC), lambda i: (0, i)),
        out_shape=jax.ShapeDtypeStruct((2, E), edge_index.dtype),
    )(edge_index)
    return out


# speedup vs baseline: 1.0431x; 1.0431x over previous
import jax
import jax.numpy as jnp
from jax.experimental import pallas as pl

_BC = 1_280_000


def _copy_block(x_ref, o_ref):
    o_ref[...] = x_ref[...]


def kernel(edge_index):
    E = edge_index.shape[1]
    out = pl.pallas_call(
        _copy_block,
        grid=(E // _BC,),
        in_specs=[pl.BlockSpec((2, _BC), lambda i: (0, i))],
        out_specs=pl.BlockSpec((2, _BC), lambda i: (0, i)),
        out_shape=jax.ShapeDtypeStruct((2, E), edge_index.dtype),
    )(edge_index)
    return out
